# baseline (speedup 1.0000x reference)
import functools

import jax
import jax.numpy as jnp
from jax.experimental import pallas as pl
from jax.experimental.pallas import tpu as pltpu


def _kc_kernel(scal_ref, x_ref, c_ref, logits_ref, c2_ref,
               ct_ref, b_ref, acc_ref, w_ref, *, nb):
    i = pl.program_id(0)
    j = pl.program_id(1)
    neg_inv_tau = scal_ref[0]
    d = acc_ref.shape[0]

    @pl.when((j == 0) & (i == 0))
    def _load_codebook():
        ct_ref[...] = jnp.transpose(c_ref[...])

    @pl.when((j == 0) & (i != 0))
    def _update_codebook():
        ct_ref[...] = acc_ref[...] / (w_ref[...] + 1e-8)

    @pl.when(j == 0)
    def _start_iter():
        ct = ct_ref[...]
        b_ref[...] = jnp.sum(ct * ct, axis=0, keepdims=True)
        acc_ref[...] = jnp.zeros_like(acc_ref)
        w_ref[...] = jnp.zeros_like(w_ref)

    x = x_ref[...]
    xc_neg2 = jnp.dot(x * (-2.0), ct_ref[...],
                      preferred_element_type=jnp.float32)
    x2 = jnp.sum(x * x, axis=1, keepdims=True)
    dist = (x2 + xc_neg2) + b_ref[...]
    logits = dist * neg_inv_tau

    last_iter = i == pl.num_programs(0) - 1

    @pl.when(last_iter)
    def _emit_logits():
        logits_ref[...] = logits

    m = jnp.max(logits, axis=1, keepdims=True)
    e = jnp.exp(logits - m)
    s = jnp.sum(e, axis=1, keepdims=True)
    assign = e / s

    w_ref[...] += jnp.sum(assign, axis=0, keepdims=True)
    acc_ref[...] += jnp.dot(jnp.transpose(x), assign,
                            preferred_element_type=jnp.float32)

    @pl.when(last_iter & (j == nb - 1))
    def _emit_centroids():
        c2_ref[...] = jnp.transpose(
            acc_ref[...] / (w_ref[...] + 1e-8))


def kernel(problem_reps, centroids, kmeans_log_tau):
    neg_inv_tau = -1.0 / jnp.exp(kmeans_log_tau)
    x = problem_reps
    n, d = x.shape
    k = centroids.shape[0]
    block_n = 4096
    nb = n // block_n
    max_iter = 2
    logits, c2 = pl.pallas_call(
        functools.partial(_kc_kernel, nb=nb),
        grid=(max_iter, nb),
        in_specs=[
            pl.BlockSpec(memory_space=pltpu.SMEM),
            pl.BlockSpec((block_n, d), lambda i, j: (j, 0)),
            pl.BlockSpec((k, d), lambda i, j: (0, 0)),
        ],
        out_specs=[
            pl.BlockSpec((block_n, k), lambda i, j: (i * j, 0)),
            pl.BlockSpec((k, d), lambda i, j: (0, 0)),
        ],
        out_shape=[
            jax.ShapeDtypeStruct((n, k), jnp.float32),
            jax.ShapeDtypeStruct((k, d), jnp.float32),
        ],
        scratch_shapes=[
            pltpu.VMEM((d, k), jnp.float32),
            pltpu.VMEM((1, k), jnp.float32),
            pltpu.VMEM((d, k), jnp.float32),
            pltpu.VMEM((1, k), jnp.float32),
        ],
    )(neg_inv_tau, x, centroids)
    return logits, c2


# speedup vs baseline: 1.0060x; 1.0060x over previous
import functools

import jax
import jax.numpy as jnp
from jax.experimental import pallas as pl
from jax.experimental.pallas import tpu as pltpu


def _kc_pass_kernel(scal_ref, x_ref, xt_ref, ct_ref, *refs, nb, emit_logits):
    if emit_logits:
        logits_ref, cout_t_ref, b_ref, acc_ref, w_ref = refs
    else:
        cout_t_ref, b_ref, acc_ref, w_ref = refs
        logits_ref = None

    d = ct_ref.shape[0]
    j = pl.program_id(0)
    neg_inv_tau = scal_ref[0]

    @pl.when(j == 0)
    def _init():
        ct = ct_ref[...]
        b_ref[...] = jnp.sum(ct * ct, axis=0, keepdims=True)
        acc_ref[...] = jnp.zeros_like(acc_ref)
        w_ref[...] = jnp.zeros_like(w_ref)

    x = x_ref[...]
    xc_neg2 = jnp.dot(x * (-2.0), ct_ref[...],
                      preferred_element_type=jnp.float32)
    x2 = jnp.sum(x * x, axis=1, keepdims=True)
    dist = (x2 + xc_neg2) + b_ref[...]
    logits = dist * neg_inv_tau
    if emit_logits:
        logits_ref[...] = logits

    m = jnp.max(logits, axis=1, keepdims=True)
    if emit_logits:
        e = jnp.exp(logits - m)
    else:
        e = jnp.exp2(logits - m)
    s = jnp.sum(e, axis=1, keepdims=True)
    rs_row = jnp.transpose(1.0 / s)

    w_ref[...] += jnp.dot(rs_row, e,
                          preferred_element_type=jnp.float32)
    acc_ref[...] += jnp.dot(xt_ref[...] * rs_row, e,
                            preferred_element_type=jnp.float32)

    @pl.when(j == nb - 1)
    def _finish():
        cout_t_ref[...] = acc_ref[...] / (w_ref[...] + 1e-8)


def _run_pass(scal, x, xt, ct, *, block_n, emit_logits, interpret=False):
    n, d = x.shape
    k = ct.shape[1]
    nb = n // block_n
    scratch = [
        pltpu.VMEM((1, k), jnp.float32),
        pltpu.VMEM((d, k), jnp.float32),
        pltpu.VMEM((1, k), jnp.float32),
    ]
    in_specs = [
        pl.BlockSpec(memory_space=pltpu.SMEM),
        pl.BlockSpec((block_n, d), lambda j: (j, 0)),
        pl.BlockSpec((d, block_n), lambda j: (0, j)),
        pl.BlockSpec((d, k), lambda j: (0, 0)),
    ]
    ct_spec = pl.BlockSpec((d, k), lambda j: (0, 0))
    ct_shape = jax.ShapeDtypeStruct((d, k), jnp.float32)
    if emit_logits:
        out_specs = [pl.BlockSpec((block_n, k), lambda j: (j, 0)), ct_spec]
        out_shape = [jax.ShapeDtypeStruct((n, k), jnp.float32), ct_shape]
    else:
        out_specs = ct_spec
        out_shape = ct_shape
    return pl.pallas_call(
        functools.partial(_kc_pass_kernel, nb=nb, emit_logits=emit_logits),
        grid=(nb,),
        in_specs=in_specs,
        out_specs=out_specs,
        out_shape=out_shape,
        scratch_shapes=scratch,
        interpret=interpret,
    )(scal, x, xt, ct)


def kernel(problem_reps, centroids, kmeans_log_tau):
    neg_inv_tau = -1.0 / jnp.exp(kmeans_log_tau)
    log2e = jnp.float32(1.4426950408889634)
    x = problem_reps
    xt = jnp.transpose(x)
    ct0 = jnp.transpose(centroids)
    block_n = 4096
    c1t = _run_pass(neg_inv_tau * log2e, x, xt, ct0,
                    block_n=block_n, emit_logits=False)
    logits, c2t = _run_pass(neg_inv_tau, x, xt, c1t,
                            block_n=block_n, emit_logits=True)
    return logits, jnp.transpose(c2t)


# speedup vs baseline: 1.1665x; 1.1595x over previous
import functools

import jax
import jax.numpy as jnp
from jax.experimental import pallas as pl
from jax.experimental.pallas import tpu as pltpu


def _kc_pass_kernel(scal_ref, x_ref, xta_ref, ct_ref, *refs, nb, emit_logits):
    if emit_logits:
        logits_ref, cout_t_ref, b_ref, acc_ref = refs
    else:
        cout_t_ref, b_ref, acc_ref = refs
        logits_ref = None

    d = ct_ref.shape[0]
    j = pl.program_id(0)
    neg_inv_tau = scal_ref[0]

    @pl.when(j == 0)
    def _init():
        ct = ct_ref[...]
        b_ref[...] = jnp.sum(ct * ct, axis=0, keepdims=True)
        acc_ref[...] = jnp.zeros_like(acc_ref)

    x = x_ref[...]
    xc_neg2 = jnp.dot(x * (-2.0), ct_ref[...],
                      preferred_element_type=jnp.float32)
    x2 = jnp.sum(x * x, axis=1, keepdims=True)
    dist = (x2 + xc_neg2) + b_ref[...]
    logits = dist * neg_inv_tau
    if emit_logits:
        logits_ref[...] = logits

    m = jnp.max(logits, axis=1, keepdims=True)
    if emit_logits:
        e = jnp.exp(logits - m)
    else:
        e = jnp.exp2(logits - m)
    s = jnp.sum(e, axis=1, keepdims=True)
    assign = e / s

    acc_ref[...] += jnp.dot(xta_ref[...], assign,
                            preferred_element_type=jnp.float32)

    @pl.when(j == nb - 1)
    def _finish():
        w = acc_ref[d:d + 1, :]
        cout_t_ref[...] = acc_ref[0:d, :] / (w + 1e-8)


def _run_pass(scal, x, xta, ct, *, block_n, emit_logits, interpret=False):
    n, d = x.shape
    k = ct.shape[1]
    nb = n // block_n
    scratch = [
        pltpu.VMEM((1, k), jnp.float32),
        pltpu.VMEM((d + 1, k), jnp.float32),
    ]
    in_specs = [
        pl.BlockSpec(memory_space=pltpu.SMEM),
        pl.BlockSpec((block_n, d), lambda j: (j, 0)),
        pl.BlockSpec((d + 1, block_n), lambda j: (0, j)),
        pl.BlockSpec((d, k), lambda j: (0, 0)),
    ]
    ct_spec = pl.BlockSpec((d, k), lambda j: (0, 0))
    ct_shape = jax.ShapeDtypeStruct((d, k), jnp.float32)
    if emit_logits:
        out_specs = [pl.BlockSpec((block_n, k), lambda j: (j, 0)), ct_spec]
        out_shape = [jax.ShapeDtypeStruct((n, k), jnp.float32), ct_shape]
    else:
        out_specs = ct_spec
        out_shape = ct_shape
    return pl.pallas_call(
        functools.partial(_kc_pass_kernel, nb=nb, emit_logits=emit_logits),
        grid=(nb,),
        in_specs=in_specs,
        out_specs=out_specs,
        out_shape=out_shape,
        scratch_shapes=scratch,
        interpret=interpret,
    )(scal, x, xta, ct)


def kernel(problem_reps, centroids, kmeans_log_tau):
    neg_inv_tau = -1.0 / jnp.exp(kmeans_log_tau)
    log2e = jnp.float32(1.4426950408889634)
    x = problem_reps
    n = x.shape[0]
    xta = jnp.concatenate(
        [jnp.transpose(x), jnp.ones((1, n), jnp.float32)], axis=0)
    ct0 = jnp.transpose(centroids)
    block_n = 4096
    c1t = _run_pass(neg_inv_tau * log2e, x, xta, ct0,
                    block_n=block_n, emit_logits=False)
    logits, c2t = _run_pass(neg_inv_tau, x, xta, c1t,
                            block_n=block_n, emit_logits=True)
    return logits, jnp.transpose(c2t)


# speedup vs baseline: 1.1817x; 1.0130x over previous
import functools

import jax
import jax.numpy as jnp
from jax.experimental import pallas as pl
from jax.experimental.pallas import tpu as pltpu


def _kc_pass_kernel(scal_ref, x_ref, xt_ref, ct_ref, *refs, nb, emit_logits):
    if emit_logits:
        logits_ref, cout_t_ref, b_ref, acc_ref, w_ref = refs
    else:
        cout_t_ref, b_ref, acc_ref, w_ref = refs
        logits_ref = None

    j = pl.program_id(0)
    neg_inv_tau = scal_ref[0]

    @pl.when(j == 0)
    def _init():
        ct = ct_ref[...]
        b_ref[...] = jnp.sum(ct * ct, axis=0, keepdims=True)
        acc_ref[...] = jnp.zeros_like(acc_ref)
        w_ref[...] = jnp.zeros_like(w_ref)

    x = x_ref[...]
    xc_neg2 = jnp.dot(x * (-2.0), ct_ref[...],
                      preferred_element_type=jnp.float32)
    x2 = jnp.sum(x * x, axis=1, keepdims=True)
    dist = (x2 + xc_neg2) + b_ref[...]
    logits = dist * neg_inv_tau
    if emit_logits:
        logits_ref[...] = logits

    m = jnp.max(logits, axis=1, keepdims=True)
    if emit_logits:
        e = jnp.exp(logits - m)
    else:
        e = jnp.exp2(logits - m)
    s = jnp.sum(e, axis=1, keepdims=True)
    assign = e / s

    w_ref[...] += jnp.sum(assign, axis=0, keepdims=True)
    acc_ref[...] += jnp.dot(xt_ref[...], assign,
                            preferred_element_type=jnp.float32)

    @pl.when(j == nb - 1)
    def _finish():
        cout_t_ref[...] = acc_ref[...] / (w_ref[...] + 1e-8)


def _run_pass(scal, x, xt, ct, *, block_n, emit_logits, interpret=False):
    n, d = x.shape
    k = ct.shape[1]
    nb = n // block_n
    scratch = [
        pltpu.VMEM((1, k), jnp.float32),
        pltpu.VMEM((d, k), jnp.float32),
        pltpu.VMEM((1, k), jnp.float32),
    ]
    in_specs = [
        pl.BlockSpec(memory_space=pltpu.SMEM),
        pl.BlockSpec((block_n, d), lambda j: (j, 0)),
        pl.BlockSpec((d, block_n), lambda j: (0, j)),
        pl.BlockSpec((d, k), lambda j: (0, 0)),
    ]
    ct_spec = pl.BlockSpec((d, k), lambda j: (0, 0))
    ct_shape = jax.ShapeDtypeStruct((d, k), jnp.float32)
    if emit_logits:
        out_specs = [pl.BlockSpec((block_n, k), lambda j: (j, 0)), ct_spec]
        out_shape = [jax.ShapeDtypeStruct((n, k), jnp.float32), ct_shape]
    else:
        out_specs = ct_spec
        out_shape = ct_shape
    return pl.pallas_call(
        functools.partial(_kc_pass_kernel, nb=nb, emit_logits=emit_logits),
        grid=(nb,),
        in_specs=in_specs,
        out_specs=out_specs,
        out_shape=out_shape,
        scratch_shapes=scratch,
        interpret=interpret,
    )(scal, x, xt, ct)


def kernel(problem_reps, centroids, kmeans_log_tau):
    neg_inv_tau = -1.0 / jnp.exp(kmeans_log_tau)
    log2e = jnp.float32(1.4426950408889634)
    x = problem_reps
    xt = jnp.transpose(x)
    ct0 = jnp.transpose(centroids)
    block_n = 4096
    c1t = _run_pass(neg_inv_tau * log2e, x, xt, ct0,
                    block_n=block_n, emit_logits=False)
    logits, c2t = _run_pass(neg_inv_tau, x, xt, c1t,
                            block_n=block_n, emit_logits=True)
    return logits, jnp.transpose(c2t)
